# baseline (speedup 1.0000x reference)
import functools

import jax
import jax.numpy as jnp
from jax import lax
from jax.experimental import pallas as pl
from jax.experimental.pallas import tpu as pltpu
from jax.experimental.pallas import tpu_sc as plsc

NC = 2
NS = 16
NW = NC * NS


_K = 96
_NBUF = 5


def _make_segsum(n_nodes, n_acc, d, n_edges_pad):
  K = _K
  e_per_w = n_edges_pad // NW
  assert e_per_w * NW == n_edges_pad and e_per_w % (K * _NBUF) == 0
  n_chunks = e_per_w // K
  rows_a = ((n_nodes // NS) + 7) // 8 * 8
  rows_last = n_nodes - (NS - 1) * rows_a
  assert rows_last > 0

  mesh = plsc.VectorSubcoreMesh(core_axis_name="c", subcore_axis_name="s")

  @functools.partial(
      pl.kernel,
      out_type=jax.ShapeDtypeStruct((NC, n_nodes, d), jnp.float32),
      mesh=mesh,
      scratch_types=[
          pltpu.VMEM((n_chunks, K), jnp.int32),
          pltpu.VMEM((n_chunks, K), jnp.int32),
          [pltpu.VMEM((K, d), jnp.float32) for _ in range(_NBUF)],
          pltpu.VMEM_SHARED((n_acc, d), jnp.float32),
          [pltpu.SemaphoreType.DMA for _ in range(_NBUF)],
          [pltpu.SemaphoreType.DMA for _ in range(_NBUF)],
      ],
      compiler_params=pltpu.CompilerParams(use_tc_tiling_on_sc=False),
  )
  def seg(y_hbm, src_hbm, dst_hbm, out_hbm, src_v, dst_v, bufs,
          acc_sh, gsems, ssems):
    c = lax.axis_index("c")
    s = lax.axis_index("s")
    wid = c * NS + s

    def zstore(t, carry):
      bufs[0][t // (d // 16), pl.ds((t % (d // 16)) * 16, 16)] = (
          jnp.zeros((16,), jnp.float32))
      return carry

    lax.fori_loop(0, K * (d // 16), zstore, 0)

    def zero_rows(base, count):
      full, rem = count // K, count % K
      for k in range(full):
        pltpu.sync_copy(bufs[0], acc_sh.at[pl.ds(base + k * K, K)])
      if rem:
        pltpu.sync_copy(bufs[0].at[pl.ds(0, rem)],
                        acc_sh.at[pl.ds(base + full * K, rem)])

    @pl.when(s < NS - 1)
    def _():
      zero_rows(s * rows_a, rows_a)

    @pl.when(s == NS - 1)
    def _():
      zero_rows((NS - 1) * rows_a, rows_last)

    pltpu.sync_copy(src_hbm.at[wid], src_v)
    pltpu.sync_copy(dst_hbm.at[wid], dst_v)
    plsc.subcore_barrier()

    for b in range(_NBUF):
      pltpu.make_async_copy(y_hbm.at[src_v.at[b]], bufs[b], gsems[b]).start()

    def round_body(i, carry):
      for b in range(_NBUF):
        j = i * _NBUF + b
        pltpu.make_async_copy(y_hbm.at[src_v.at[j]], bufs[b], gsems[b]).wait()
        pltpu.sync_copy(bufs[b], acc_sh.at[dst_v.at[j]], add=True)

        @pl.when(j + _NBUF < n_chunks)
        def _():
          pltpu.make_async_copy(y_hbm.at[src_v.at[j + _NBUF]], bufs[b],
                                gsems[b]).start()

      return carry

    lax.fori_loop(0, n_chunks // _NBUF, round_body, 0)
    plsc.subcore_barrier()

    @pl.when(s < NS - 1)
    def _():
      pltpu.sync_copy(acc_sh.at[pl.ds(s * rows_a, rows_a)],
                      out_hbm.at[c, pl.ds(s * rows_a, rows_a)])

    @pl.when(s == NS - 1)
    def _():
      pltpu.sync_copy(acc_sh.at[pl.ds((NS - 1) * rows_a, rows_last)],
                      out_hbm.at[c, pl.ds((NS - 1) * rows_a, rows_last)])

  return seg


def _mm_body(x_ref, w_ref, o_ref):
  o_ref[...] = jnp.dot(x_ref[...], w_ref[...],
                       preferred_element_type=jnp.float32)


def _tc2_body(y_ref, p0_ref, p1_ref, ba_ref, wb_ref, bb_ref, wn_ref, o_ref):
  t = jnp.maximum(y_ref[...] + p0_ref[...] + p1_ref[...] + ba_ref[...], 0.0)
  h = jnp.dot(t, wb_ref[...], preferred_element_type=jnp.float32) + bb_ref[...]
  h = jnp.maximum(h, 0.0)
  o_ref[...] = jnp.dot(h, wn_ref[...], preferred_element_type=jnp.float32)


def _tc3_body(y_ref, p0_ref, p1_ref, ba_ref, wb_ref, bb_ref, wl_ref, bl_ref,
              o_ref):
  t = jnp.maximum(y_ref[...] + p0_ref[...] + p1_ref[...] + ba_ref[...], 0.0)
  h = jnp.dot(t, wb_ref[...], preferred_element_type=jnp.float32) + bb_ref[...]
  h = jnp.maximum(h, 0.0)
  logits = (jnp.dot(h, wl_ref[...], preferred_element_type=jnp.float32)
            + bl_ref[...])
  m = jnp.max(logits, axis=1, keepdims=True)
  lse = m + jnp.log(jnp.sum(jnp.exp(logits - m), axis=1, keepdims=True))
  o_ref[...] = logits - lse


def _row_blocked(body, n, blk, in_shapes, out_cols):
  grid = n // blk
  in_specs = []
  for shp in in_shapes:
    if shp[0] == n:
      in_specs.append(
          pl.BlockSpec((blk,) + shp[1:],
                       lambda i, r=len(shp) - 1: (i,) + (0,) * r))
    else:
      in_specs.append(pl.BlockSpec(shp, lambda i, r=len(shp): (0,) * r))
  return pl.pallas_call(
      body,
      grid=(grid,),
      in_specs=in_specs,
      out_specs=pl.BlockSpec((blk, out_cols), lambda i: (i, 0)),
      out_shape=jax.ShapeDtypeStruct((n, out_cols), jnp.float32),
  )


def kernel(x, edge_index, W1a, b1a, W1b, b1b, W2a, b2a, W2b, b2b, Wl, bl):
  n, d_in = x.shape
  hidden = W1a.shape[1]
  n_classes = Wl.shape[1]
  n_edges = edge_index.shape[1]
  unit = NW * _K * _NBUF
  n_edges_pad = ((n_edges + unit - 1) // unit) * unit
  pad = n_edges_pad - n_edges

  src = edge_index[0].astype(jnp.int32)
  dst = edge_index[1].astype(jnp.int32)
  if pad:
    src = jnp.concatenate([src, jnp.zeros((pad,), jnp.int32)])
    dst = jnp.concatenate(
        [dst, n + (jnp.arange(pad, dtype=jnp.int32) % 128)])
  src3 = src.reshape(NW, -1, _K)
  dst3 = dst.reshape(NW, -1, _K)
  b1a_r = b1a.reshape(1, hidden)
  b1b_r = b1b.reshape(1, hidden)
  b2a_r = b2a.reshape(1, hidden)
  b2b_r = b2b.reshape(1, hidden)
  bl_r = bl.reshape(1, n_classes)

  segsum = _make_segsum(n, n + 128, hidden, n_edges_pad)
  blk = 1000

  y1 = _row_blocked(_mm_body, n, blk, [(n, d_in), (d_in, hidden)], hidden)(
      x, W1a)
  p1 = segsum(y1, src3, dst3)
  y2 = _row_blocked(
      _tc2_body, n, blk,
      [(n, hidden), (n, hidden), (n, hidden), (1, hidden), (hidden, hidden),
       (1, hidden), (hidden, hidden)], hidden)(
           y1, p1[0], p1[1], b1a_r, W1b, b1b_r, W2a)

  p2 = segsum(y2, src3, dst3)
  out = _row_blocked(
      _tc3_body, n, blk,
      [(n, hidden), (n, hidden), (n, hidden), (1, hidden), (hidden, hidden),
       (1, hidden), (hidden, n_classes), (1, n_classes)], n_classes)(
           y2, p2[0], p2[1], b2a_r, W2b, b2b_r, Wl, bl_r)
  return out


# speedup vs baseline: 1.5047x; 1.5047x over previous
import functools

import jax
import jax.numpy as jnp
from jax import lax
from jax.experimental import pallas as pl
from jax.experimental.pallas import tpu as pltpu
from jax.experimental.pallas import tpu_sc as plsc

NC = 2
NS = 16
NW = NC * NS


_K = 80
_NBUF = 5


def _make_segsum(n_nodes, n_acc, d, n_edges_pad):
  K = _K
  e_per_w = n_edges_pad // NW
  assert e_per_w * NW == n_edges_pad and e_per_w % (K * _NBUF) == 0
  n_chunks = e_per_w // K
  rows_a = ((n_nodes // NS) + 7) // 8 * 8
  rows_last = n_nodes - (NS - 1) * rows_a
  assert rows_last > 0

  mesh = plsc.VectorSubcoreMesh(core_axis_name="c", subcore_axis_name="s")

  @functools.partial(
      pl.kernel,
      out_type=jax.ShapeDtypeStruct((NC, n_nodes, d), jnp.float32),
      mesh=mesh,
      scratch_types=[
          pltpu.VMEM((n_chunks, K), jnp.int32),
          pltpu.VMEM((n_chunks, K), jnp.int32),
          [pltpu.VMEM((K, d), jnp.float32) for _ in range(_NBUF)],
          pltpu.VMEM_SHARED((n_acc, d), jnp.float32),
          [pltpu.SemaphoreType.DMA for _ in range(_NBUF)],
          [pltpu.SemaphoreType.DMA for _ in range(_NBUF)],
      ],
      compiler_params=pltpu.CompilerParams(use_tc_tiling_on_sc=False),
  )
  def seg(y_hbm, src_hbm, dst_hbm, out_hbm, src_v, dst_v, bufs,
          acc_sh, gsems, ssems):
    c = lax.axis_index("c")
    s = lax.axis_index("s")
    wid = c * NS + s

    def zstore(t, carry):
      bufs[0][t // (d // 16), pl.ds((t % (d // 16)) * 16, 16)] = (
          jnp.zeros((16,), jnp.float32))
      return carry

    lax.fori_loop(0, K * (d // 16), zstore, 0)

    def zero_rows(base, count):
      full, rem = count // K, count % K
      for k in range(full):
        pltpu.sync_copy(bufs[0], acc_sh.at[pl.ds(base + k * K, K)])
      if rem:
        pltpu.sync_copy(bufs[0].at[pl.ds(0, rem)],
                        acc_sh.at[pl.ds(base + full * K, rem)])

    @pl.when(s < NS - 1)
    def _():
      zero_rows(s * rows_a, rows_a)

    @pl.when(s == NS - 1)
    def _():
      zero_rows((NS - 1) * rows_a, rows_last)

    pltpu.sync_copy(src_hbm.at[wid], src_v)
    pltpu.sync_copy(dst_hbm.at[wid], dst_v)
    plsc.subcore_barrier()

    for b in range(_NBUF):
      pltpu.make_async_copy(y_hbm.at[src_v.at[b]], bufs[b], gsems[b]).start()

    def round_body(i, carry):
      for b in range(_NBUF):
        j = i * _NBUF + b
        pltpu.make_async_copy(y_hbm.at[src_v.at[j]], bufs[b], gsems[b]).wait()
        pltpu.sync_copy(bufs[b], acc_sh.at[dst_v.at[j]], add=True)

        @pl.when(j + _NBUF < n_chunks)
        def _():
          pltpu.make_async_copy(y_hbm.at[src_v.at[j + _NBUF]], bufs[b],
                                gsems[b]).start()

      return carry

    lax.fori_loop(0, n_chunks // _NBUF, round_body, 0)
    plsc.subcore_barrier()

    @pl.when(s < NS - 1)
    def _():
      pltpu.sync_copy(acc_sh.at[pl.ds(s * rows_a, rows_a)],
                      out_hbm.at[c, pl.ds(s * rows_a, rows_a)])

    @pl.when(s == NS - 1)
    def _():
      pltpu.sync_copy(acc_sh.at[pl.ds((NS - 1) * rows_a, rows_last)],
                      out_hbm.at[c, pl.ds((NS - 1) * rows_a, rows_last)])

  return seg


def _mm_body(x_ref, w_ref, o_ref):
  o_ref[...] = jnp.dot(x_ref[...], w_ref[...],
                       preferred_element_type=jnp.float32)


def _tc2_body(y_ref, p0_ref, p1_ref, ba_ref, wb_ref, bb_ref, wn_ref, o_ref):
  t = jnp.maximum(y_ref[...] + p0_ref[...] + p1_ref[...] + ba_ref[...], 0.0)
  h = jnp.dot(t, wb_ref[...], preferred_element_type=jnp.float32) + bb_ref[...]
  h = jnp.maximum(h, 0.0)
  o_ref[...] = jnp.dot(h, wn_ref[...], preferred_element_type=jnp.float32)


def _tc3_body(y_ref, p0_ref, p1_ref, ba_ref, wb_ref, bb_ref, wl_ref, bl_ref,
              o_ref):
  t = jnp.maximum(y_ref[...] + p0_ref[...] + p1_ref[...] + ba_ref[...], 0.0)
  h = jnp.dot(t, wb_ref[...], preferred_element_type=jnp.float32) + bb_ref[...]
  h = jnp.maximum(h, 0.0)
  logits = (jnp.dot(h, wl_ref[...], preferred_element_type=jnp.float32)
            + bl_ref[...])
  m = jnp.max(logits, axis=1, keepdims=True)
  lse = m + jnp.log(jnp.sum(jnp.exp(logits - m), axis=1, keepdims=True))
  o_ref[...] = logits - lse


def _row_blocked(body, n, blk, in_shapes, out_cols):
  grid = n // blk
  in_specs = []
  for shp in in_shapes:
    if shp[0] == n:
      in_specs.append(
          pl.BlockSpec((blk,) + shp[1:],
                       lambda i, r=len(shp) - 1: (i,) + (0,) * r))
    else:
      in_specs.append(pl.BlockSpec(shp, lambda i, r=len(shp): (0,) * r))
  return pl.pallas_call(
      body,
      grid=(grid,),
      in_specs=in_specs,
      out_specs=pl.BlockSpec((blk, out_cols), lambda i: (i, 0)),
      out_shape=jax.ShapeDtypeStruct((n, out_cols), jnp.float32),
  )


def kernel(x, edge_index, W1a, b1a, W1b, b1b, W2a, b2a, W2b, b2b, Wl, bl):
  n, d_in = x.shape
  hidden = W1a.shape[1]
  n_classes = Wl.shape[1]
  n_edges = edge_index.shape[1]
  unit = NW * _K * _NBUF
  n_edges_pad = ((n_edges + unit - 1) // unit) * unit
  pad = n_edges_pad - n_edges

  src = edge_index[0].astype(jnp.int32)
  dst = edge_index[1].astype(jnp.int32)
  if pad:
    src = jnp.concatenate([src, jnp.zeros((pad,), jnp.int32)])
    dst = jnp.concatenate(
        [dst, n + (jnp.arange(pad, dtype=jnp.int32) % 128)])
  src3 = src.reshape(NW, -1, _K)
  dst3 = dst.reshape(NW, -1, _K)
  b1a_r = b1a.reshape(1, hidden)
  b1b_r = b1b.reshape(1, hidden)
  b2a_r = b2a.reshape(1, hidden)
  b2b_r = b2b.reshape(1, hidden)
  bl_r = bl.reshape(1, n_classes)

  segsum = _make_segsum(n, n + 128, hidden, n_edges_pad)
  blk = 1000

  y1 = _row_blocked(_mm_body, n, blk, [(n, d_in), (d_in, hidden)], hidden)(
      x, W1a)
  p1 = segsum(y1, src3, dst3)
  y2 = _row_blocked(
      _tc2_body, n, blk,
      [(n, hidden), (n, hidden), (n, hidden), (1, hidden), (hidden, hidden),
       (1, hidden), (hidden, hidden)], hidden)(
           y1, p1[0], p1[1], b1a_r, W1b, b1b_r, W2a)

  p2 = segsum(y2, src3, dst3)
  out = _row_blocked(
      _tc3_body, n, blk,
      [(n, hidden), (n, hidden), (n, hidden), (1, hidden), (hidden, hidden),
       (1, hidden), (hidden, n_classes), (1, n_classes)], n_classes)(
           y2, p2[0], p2[1], b2a_r, W2b, b2b_r, Wl, bl_r)
  return out
